# Initial kernel scaffold; baseline (speedup 1.0000x reference)
import jax
import jax.numpy as jnp
from jax.experimental import pallas as pl


def kernel(img1, img2, w):
    raise NotImplementedError("write your pallas kernel here")


# speedup vs baseline: 1.5473x; 1.5473x over previous
import functools

import jax
import jax.numpy as jnp
from jax.experimental import pallas as pl
from jax.experimental.pallas import tpu as pltpu


def _fused_depth_kernel(w_ref, img1_ref, img2_ref, o1_ref, o2_ref, lp_ref, *, C):
    bias = w_ref[C]

    def proxy_depth(img_ref):
        acc = None
        for c in range(C):
            xc = img_ref[0, c].astype(jnp.float32)
            term = w_ref[c] * jnp.clip(xc, -1.0, 1.0)
            acc = term if acc is None else acc + term
        return acc + bias

    def min_max_normalize(d):
        mn = jnp.min(d)
        mx = jnp.max(d)
        ok = mx > mn
        scale = jnp.where(ok, 1.0 / jnp.where(ok, mx - mn, 1.0), 1.0)
        off = jnp.where(ok, -mn * scale, 0.0)
        return d * scale + off

    n1 = min_max_normalize(proxy_depth(img1_ref))
    n2 = min_max_normalize(proxy_depth(img2_ref))
    o1_ref[0] = n1
    o2_ref[0] = n2
    lp_ref[0] = jnp.sum(jnp.abs(n1 - n2), axis=0, keepdims=True)


def kernel(img1, img2, w):
    B, C, H, W = img1.shape
    N = H * W

    w = w.astype(jnp.float32)
    wvec = jnp.concatenate([127.5 * w, 127.5 * jnp.sum(w)[None]])

    depth_shape = jax.ShapeDtypeStruct((B, H, W), jnp.float32)
    lp_shape = jax.ShapeDtypeStruct((B, 1, W), jnp.float32)

    img_bytes = int(img1.dtype.itemsize)
    ce = pl.CostEstimate(
        flops=10 * B * C * N,
        transcendentals=0,
        bytes_accessed=2 * B * C * N * img_bytes + 2 * B * N * 4,
    )

    d1n, d2n, lp = pl.pallas_call(
        functools.partial(_fused_depth_kernel, C=C),
        out_shape=(depth_shape, depth_shape, lp_shape),
        grid_spec=pltpu.PrefetchScalarGridSpec(
            num_scalar_prefetch=0,
            grid=(B,),
            in_specs=[
                pl.BlockSpec(memory_space=pltpu.MemorySpace.SMEM),
                pl.BlockSpec((1, C, H, W), lambda b: (b, 0, 0, 0)),
                pl.BlockSpec((1, C, H, W), lambda b: (b, 0, 0, 0)),
            ],
            out_specs=[
                pl.BlockSpec((1, H, W), lambda b: (b, 0, 0)),
                pl.BlockSpec((1, H, W), lambda b: (b, 0, 0)),
                pl.BlockSpec((1, 1, W), lambda b: (b, 0, 0)),
            ],
        ),
        compiler_params=pltpu.CompilerParams(
            dimension_semantics=("parallel",),
            vmem_limit_bytes=64 * 1024 * 1024,
        ),
        cost_estimate=ce,
    )(wvec, img1, img2)

    loss = lp.sum() / jnp.float32(B * N)
    return loss, d1n.reshape(B, 1, H, W), d2n.reshape(B, 1, H, W)
